# Initial kernel scaffold; baseline (speedup 1.0000x reference)
import jax
import jax.numpy as jnp
from jax.experimental import pallas as pl


def kernel(hidden_states, connections, q_w, k_w_must, v_w_must, k_w_may, v_w_may, k_w_next, v_w_next, o_w):
    raise NotImplementedError("write your pallas kernel here")


# speedup vs baseline: 2.0555x; 2.0555x over previous
import functools

import jax
import jax.numpy as jnp
from jax import lax
from jax.experimental import pallas as pl
from jax.experimental.pallas import tpu as pltpu
from jax.experimental.pallas import tpu_sc as plsc

_B, _S, _D = 2, 4096, 1024
_NH, _HD, _NC = 16, 64, 4
_P = _B * _S
_NW = 32
_PER_W = _P // _NW
_G = 16
_NG = _PER_W // _G


def _proj_body(a_ref, *refs):
    w_refs, o_refs = refs[:7], refs[7:]
    a = a_ref[...].astype(jnp.bfloat16)
    for w, o in zip(w_refs, o_refs):
        o[...] = jnp.dot(a, w[...], preferred_element_type=jnp.float32)


def _project(h2, ws):
    bm = 256
    return pl.pallas_call(
        _proj_body,
        grid=(_P // bm,),
        in_specs=[pl.BlockSpec((bm, _D), lambda i: (i, 0))]
        + [pl.BlockSpec((_D, _D), lambda i: (0, 0))] * 7,
        out_specs=[pl.BlockSpec((bm, _D), lambda i: (i, 0))] * 7,
        out_shape=[jax.ShapeDtypeStruct((_P, _D), jnp.float32)] * 7,
    )(h2, *ws)


def _sc_attn_body(q_hbm, idx_hbm, k0, k1, k2, v0, v1, v2, out_hbm,
                  q_v, idx_v, b0, b1, b2, b3, out_v, w_scr, sem):
    cid = lax.axis_index("c")
    sid = lax.axis_index("s")
    wid = sid * 2 + cid
    base = wid * _PER_W

    pltpu.sync_copy(idx_hbm.at[:, pl.ds(base, _PER_W)], idx_v)

    ktabs = (k0, k1, k1, k2)
    vtabs = (v0, v1, v1, v2)
    bufs = (b0, b1, b2, b3)

    def group(g, carry):
        gb = base + g * _G
        pltpu.sync_copy(q_hbm.at[pl.ds(gb, _G)], q_v)

        cps = []
        for c in range(_NC):
            idxc = idx_v[c, pl.ds(g * _G, _G)]
            cp = pltpu.make_async_copy(ktabs[c].at[idxc], bufs[c], sem)
            cp.start()
            cps.append(cp)
        for cp in cps:
            cp.wait()

        def posk(p, pc):
            sl0 = pl.ds(0, 16)
            qv = q_v[p, sl0]
            acc = [qv * bufs[c][p, sl0] for c in range(_NC)]
            for d in range(1, _HD):
                sl = pl.ds(d * 16, 16)
                qv = q_v[p, sl]
                for c in range(_NC):
                    acc[c] = acc[c] + qv * bufs[c][p, sl]
            m = jnp.maximum(jnp.maximum(acc[0], acc[1]),
                            jnp.maximum(acc[2], acc[3]))
            es = [jnp.exp(a - m) for a in acc]
            r = 1.0 / ((es[0] + es[1]) + (es[2] + es[3]))
            for c in range(_NC):
                w_scr[pl.ds(p * 64 + c * 16, 16)] = es[c] * r
            return pc

        lax.fori_loop(0, _G, posk, 0)

        cps = []
        for c in range(_NC):
            idxc = idx_v[c, pl.ds(g * _G, _G)]
            cp = pltpu.make_async_copy(vtabs[c].at[idxc], bufs[c], sem)
            cp.start()
            cps.append(cp)
        for cp in cps:
            cp.wait()

        def posv(p, pc):
            ws = [w_scr[pl.ds(p * 64 + c * 16, 16)] for c in range(_NC)]
            for d in range(_HD):
                sl = pl.ds(d * 16, 16)
                o = ws[0] * bufs[0][p, sl]
                for c in range(1, _NC):
                    o = o + ws[c] * bufs[c][p, sl]
                out_v[p, sl] = o
            return pc

        lax.fori_loop(0, _G, posv, 0)
        pltpu.sync_copy(out_v, out_hbm.at[pl.ds(gb, _G)])
        return carry

    lax.fori_loop(0, _NG, group, 0)


_sc_attn = pl.kernel(
    _sc_attn_body,
    out_type=jax.ShapeDtypeStruct((_P, _D), jnp.float32),
    mesh=plsc.VectorSubcoreMesh(core_axis_name="c", subcore_axis_name="s",
                                num_cores=2, num_subcores=16),
    scratch_types=[
        pltpu.VMEM((_G, _D), jnp.float32),
        pltpu.VMEM((_NC, _PER_W), jnp.int32),
        pltpu.VMEM((_G, _D), jnp.float32),
        pltpu.VMEM((_G, _D), jnp.float32),
        pltpu.VMEM((_G, _D), jnp.float32),
        pltpu.VMEM((_G, _D), jnp.float32),
        pltpu.VMEM((_G, _D), jnp.float32),
        pltpu.VMEM((_G * 64,), jnp.float32),
        pltpu.SemaphoreType.DMA,
    ],
)


def _out_body(a_ref, w_ref, o_ref):
    o_ref[...] = jnp.dot(a_ref[...].astype(jnp.bfloat16), w_ref[...],
                         preferred_element_type=jnp.float32)


def _outproj(attn, o_w_bf16):
    bm = 256
    return pl.pallas_call(
        _out_body,
        grid=(_P // bm,),
        in_specs=[pl.BlockSpec((bm, _D), lambda i: (i, 0)),
                  pl.BlockSpec((_D, _D), lambda i: (0, 0))],
        out_specs=pl.BlockSpec((bm, _D), lambda i: (i, 0)),
        out_shape=jax.ShapeDtypeStruct((_P, _D), jnp.float32),
    )(attn, o_w_bf16)


def kernel(hidden_states, connections, q_w, k_w_must, v_w_must, k_w_may,
           v_w_may, k_w_next, v_w_next, o_w):
    h2 = hidden_states.reshape(_P, _D)
    scale = 1.0 / (_HD ** 0.5)
    perm = (jnp.arange(_D) % _NH) * _HD + jnp.arange(_D) // _NH
    ws = [
        (q_w * scale)[:, perm].astype(jnp.bfloat16),
        k_w_must[:, perm].astype(jnp.bfloat16),
        k_w_may[:, perm].astype(jnp.bfloat16),
        k_w_next[:, perm].astype(jnp.bfloat16),
        v_w_must[:, perm].astype(jnp.bfloat16),
        v_w_may[:, perm].astype(jnp.bfloat16),
        v_w_next[:, perm].astype(jnp.bfloat16),
    ]
    q, km, ka, kn, vm, va, vn = _project(h2, ws)
    conn = connections.astype(jnp.int32) + (
        jnp.arange(_B, dtype=jnp.int32) * _S)[:, None, None]
    idx = conn.reshape(_P, _NC).T
    attn = _sc_attn(q, idx, km, ka, kn, vm, va, vn)
    out = _outproj(attn, o_w[perm, :].astype(jnp.bfloat16))
    return out.reshape(_B, _S, _D)


# speedup vs baseline: 2.6178x; 1.2736x over previous
import jax
import jax.numpy as jnp
from jax import lax
from jax.experimental import pallas as pl
from jax.experimental.pallas import tpu as pltpu
from jax.experimental.pallas import tpu_sc as plsc

_B, _S, _D = 2, 4096, 1024
_NH, _HD, _NC = 16, 64, 4
_P = _B * _S
_NW = 32
_PER_W = _P // _NW
_G = 8
_NG = _PER_W // _G


def _proj_body(a_ref, *refs):
    w_refs, o_refs = refs[:7], refs[7:]
    a = a_ref[...].astype(jnp.bfloat16)
    for w, o in zip(w_refs, o_refs):
        o[...] = jnp.dot(a, w[...], preferred_element_type=jnp.float32)


def _project(h2, ws):
    bm = 256
    return pl.pallas_call(
        _proj_body,
        grid=(_P // bm,),
        in_specs=[pl.BlockSpec((bm, _D), lambda i: (i, 0))]
        + [pl.BlockSpec((_D, _D), lambda i: (0, 0))] * 7,
        out_specs=[pl.BlockSpec((bm, _D), lambda i: (i, 0))] * 7,
        out_shape=[jax.ShapeDtypeStruct((_P, _D), jnp.float32)] * 7,
    )(h2, *ws)


def _sc_attn_body(q_hbm, idx_hbm, k0, k1, k2, v0, v1, v2, out_hbm,
                  q_v, idx_v, kb0, kb1, kb2, kb3, vb0, vb1, vb2, vb3,
                  out_v, w_scr, semq, semk, semv):
    cid = lax.axis_index("c")
    sid = lax.axis_index("s")
    wid = sid * 2 + cid
    base = wid * _PER_W

    pltpu.sync_copy(idx_hbm.at[:, pl.ds(base, _PER_W)], idx_v)

    ktabs = (k0, k1, k1, k2)
    vtabs = (v0, v1, v1, v2)
    kbufs = (kb0, kb1, kb2, kb3)
    vbufs = (vb0, vb1, vb2, vb3)

    def start_k(g):
        gb = base + g * _G
        pltpu.make_async_copy(q_hbm.at[pl.ds(gb, _G)], q_v, semq).start()
        for c in range(_NC):
            idxc = idx_v.at[c, pl.ds(g * _G, _G)]
            pltpu.make_async_copy(ktabs[c].at[idxc], kbufs[c], semk).start()

    def wait_k():
        pltpu.make_async_copy(q_hbm.at[pl.ds(0, _G)], q_v, semq).wait()
        for c in range(_NC):
            pltpu.make_async_copy(ktabs[c].at[idx_v.at[c, pl.ds(0, _G)]],
                                  kbufs[c], semk).wait()

    def start_v(g):
        for c in range(_NC):
            idxc = idx_v.at[c, pl.ds(g * _G, _G)]
            pltpu.make_async_copy(vtabs[c].at[idxc], vbufs[c], semv).start()

    def wait_v():
        for c in range(_NC):
            pltpu.make_async_copy(vtabs[c].at[idx_v.at[c, pl.ds(0, _G)]],
                                  vbufs[c], semv).wait()

    start_k(0)

    def group(g, carry):
        wait_k()
        start_v(g)

        def posk(p, pc):
            qv = q_v[p, pl.ds(0, 16)]
            acc = [qv * kbufs[c][p, pl.ds(0, 16)] for c in range(_NC)]
            for d in range(1, _HD):
                sl = pl.ds(d * 16, 16)
                qv = q_v[p, sl]
                for c in range(_NC):
                    acc[c] = acc[c] + qv * kbufs[c][p, sl]
            m = jnp.maximum(jnp.maximum(acc[0], acc[1]),
                            jnp.maximum(acc[2], acc[3]))
            es = [jnp.exp(a - m) for a in acc]
            r = 1.0 / ((es[0] + es[1]) + (es[2] + es[3]))
            for c in range(_NC):
                w_scr[pl.ds(p * 64 + c * 16, 16)] = es[c] * r
            return pc

        lax.fori_loop(0, _G, posk, 0)

        wait_v()

        @pl.when(g + 1 < _NG)
        def _():
            start_k(g + 1)

        def posv(p, pc):
            ws = [w_scr[pl.ds(p * 64 + c * 16, 16)] for c in range(_NC)]
            for d in range(_HD):
                sl = pl.ds(d * 16, 16)
                o = ws[0] * vbufs[0][p, sl]
                for c in range(1, _NC):
                    o = o + ws[c] * vbufs[c][p, sl]
                out_v[p, sl] = o
            return pc

        lax.fori_loop(0, _G, posv, 0)
        gb = base + g * _G
        pltpu.sync_copy(out_v, out_hbm.at[pl.ds(gb, _G)])
        return carry

    lax.fori_loop(0, _NG, group, 0)


_sc_attn = pl.kernel(
    _sc_attn_body,
    out_type=jax.ShapeDtypeStruct((_P, _D), jnp.float32),
    mesh=plsc.VectorSubcoreMesh(core_axis_name="c", subcore_axis_name="s",
                                num_cores=2, num_subcores=16),
    scratch_types=[
        pltpu.VMEM((_G, _D), jnp.float32),
        pltpu.VMEM((_NC, _PER_W), jnp.int32),
        pltpu.VMEM((_G, _D), jnp.float32),
        pltpu.VMEM((_G, _D), jnp.float32),
        pltpu.VMEM((_G, _D), jnp.float32),
        pltpu.VMEM((_G, _D), jnp.float32),
        pltpu.VMEM((_G, _D), jnp.float32),
        pltpu.VMEM((_G, _D), jnp.float32),
        pltpu.VMEM((_G, _D), jnp.float32),
        pltpu.VMEM((_G, _D), jnp.float32),
        pltpu.VMEM((_G, _D), jnp.float32),
        pltpu.VMEM((_G * 64,), jnp.float32),
        pltpu.SemaphoreType.DMA,
        pltpu.SemaphoreType.DMA,
        pltpu.SemaphoreType.DMA,
    ],
)


def _out_body(a_ref, w_ref, o_ref):
    o_ref[...] = jnp.dot(a_ref[...].astype(jnp.bfloat16), w_ref[...],
                         preferred_element_type=jnp.float32)


def _outproj(attn, o_w_bf16):
    bm = 256
    return pl.pallas_call(
        _out_body,
        grid=(_P // bm,),
        in_specs=[pl.BlockSpec((bm, _D), lambda i: (i, 0)),
                  pl.BlockSpec((_D, _D), lambda i: (0, 0))],
        out_specs=pl.BlockSpec((bm, _D), lambda i: (i, 0)),
        out_shape=jax.ShapeDtypeStruct((_P, _D), jnp.float32),
    )(attn, o_w_bf16)


def kernel(hidden_states, connections, q_w, k_w_must, v_w_must, k_w_may,
           v_w_may, k_w_next, v_w_next, o_w):
    h2 = hidden_states.reshape(_P, _D)
    scale = 1.0 / (_HD ** 0.5)
    i = jnp.arange(_D)
    perm = (i % _NH) * _HD + i // _NH
    ws = [
        (q_w * scale)[:, perm].astype(jnp.bfloat16),
        k_w_must[:, perm].astype(jnp.bfloat16),
        k_w_may[:, perm].astype(jnp.bfloat16),
        k_w_next[:, perm].astype(jnp.bfloat16),
        v_w_must[:, perm].astype(jnp.bfloat16),
        v_w_may[:, perm].astype(jnp.bfloat16),
        v_w_next[:, perm].astype(jnp.bfloat16),
    ]
    q, km, ka, kn, vm, va, vn = _project(h2, ws)
    conn = connections.astype(jnp.int32) + (
        jnp.arange(_B, dtype=jnp.int32) * _S)[:, None, None]
    idx = conn.reshape(_P, _NC).T
    attn = _sc_attn(q, idx, km, ka, kn, vm, va, vn)
    out = _outproj(attn, o_w[perm, :].astype(jnp.bfloat16))
    return out.reshape(_B, _S, _D)


# speedup vs baseline: 2.6275x; 1.0037x over previous
import jax
import jax.numpy as jnp
from jax import lax
from jax.experimental import pallas as pl
from jax.experimental.pallas import tpu as pltpu
from jax.experimental.pallas import tpu_sc as plsc

_B, _S, _D = 2, 4096, 1024
_NH, _HD, _NC = 16, 64, 4
_P = _B * _S
_P2 = _S
_NW = 32
_PER_W = _P2 // _NW
_G = 8
_NG = _PER_W // _G


def _proj_body(a_ref, *refs):
    w_refs, o_refs = refs[:7], refs[7:]
    a = a_ref[...].astype(jnp.bfloat16)
    for w, o in zip(w_refs, o_refs):
        o[...] = jnp.dot(a, w[...], preferred_element_type=jnp.float32)


def _project_half(h2, ws, b):
    bm = 256
    nsteps = _P2 // bm
    return pl.pallas_call(
        _proj_body,
        grid=(nsteps,),
        in_specs=[pl.BlockSpec((bm, _D), lambda i, _b=b: (i + _b * (_P2 // bm), 0))]
        + [pl.BlockSpec((_D, _D), lambda i: (0, 0))] * 7,
        out_specs=[pl.BlockSpec((bm, _D), lambda i: (i, 0))] * 7,
        out_shape=[jax.ShapeDtypeStruct((_P2, _D), jnp.float32)] * 7,
    )(h2, *ws)


def _sc_attn_body(q_hbm, idx_hbm, k0, k1, k2, v0, v1, v2, out_hbm,
                  q_v, idx_v, kb0, kb1, kb2, kb3, vb0, vb1, vb2, vb3,
                  out_v, w_scr, semq, semk, semv):
    cid = lax.axis_index("c")
    sid = lax.axis_index("s")
    wid = sid * 2 + cid
    base = wid * _PER_W

    pltpu.sync_copy(idx_hbm.at[:, pl.ds(base, _PER_W)], idx_v)

    ktabs = (k0, k1, k1, k2)
    vtabs = (v0, v1, v1, v2)
    kbufs = (kb0, kb1, kb2, kb3)
    vbufs = (vb0, vb1, vb2, vb3)

    def start_k(g):
        gb = base + g * _G
        pltpu.make_async_copy(q_hbm.at[pl.ds(gb, _G)], q_v, semq).start()
        for c in range(_NC):
            idxc = idx_v.at[c, pl.ds(g * _G, _G)]
            pltpu.make_async_copy(ktabs[c].at[idxc], kbufs[c], semk).start()

    def wait_k():
        pltpu.make_async_copy(q_hbm.at[pl.ds(0, _G)], q_v, semq).wait()
        for c in range(_NC):
            pltpu.make_async_copy(ktabs[c].at[idx_v.at[c, pl.ds(0, _G)]],
                                  kbufs[c], semk).wait()

    def start_v(g):
        for c in range(_NC):
            idxc = idx_v.at[c, pl.ds(g * _G, _G)]
            pltpu.make_async_copy(vtabs[c].at[idxc], vbufs[c], semv).start()

    def wait_v():
        for c in range(_NC):
            pltpu.make_async_copy(vtabs[c].at[idx_v.at[c, pl.ds(0, _G)]],
                                  vbufs[c], semv).wait()

    start_k(0)

    def group(g, carry):
        wait_k()
        start_v(g)

        def posk(p, pc):
            qv = q_v[p, pl.ds(0, 16)]
            acc = [qv * kbufs[c][p, pl.ds(0, 16)] for c in range(_NC)]
            for d in range(1, _HD):
                sl = pl.ds(d * 16, 16)
                qv = q_v[p, sl]
                for c in range(_NC):
                    acc[c] = acc[c] + qv * kbufs[c][p, sl]
            m = jnp.maximum(jnp.maximum(acc[0], acc[1]),
                            jnp.maximum(acc[2], acc[3]))
            es = [jnp.exp(a - m) for a in acc]
            r = 1.0 / ((es[0] + es[1]) + (es[2] + es[3]))
            for c in range(_NC):
                w_scr[pl.ds(p * 64 + c * 16, 16)] = es[c] * r
            return pc

        lax.fori_loop(0, _G, posk, 0)

        wait_v()

        @pl.when(g + 1 < _NG)
        def _():
            start_k(g + 1)

        def posv(p, pc):
            ws = [w_scr[pl.ds(p * 64 + c * 16, 16)] for c in range(_NC)]
            for d in range(_HD):
                sl = pl.ds(d * 16, 16)
                o = ws[0] * vbufs[0][p, sl]
                for c in range(1, _NC):
                    o = o + ws[c] * vbufs[c][p, sl]
                out_v[p, sl] = o
            return pc

        lax.fori_loop(0, _G, posv, 0)
        gb = base + g * _G
        pltpu.sync_copy(out_v, out_hbm.at[pl.ds(gb, _G)])
        return carry

    lax.fori_loop(0, _NG, group, 0)


_sc_attn = pl.kernel(
    _sc_attn_body,
    out_type=jax.ShapeDtypeStruct((_P2, _D), jnp.float32),
    mesh=plsc.VectorSubcoreMesh(core_axis_name="c", subcore_axis_name="s",
                                num_cores=2, num_subcores=16),
    scratch_types=[
        pltpu.VMEM((_G, _D), jnp.float32),
        pltpu.VMEM((_NC, _PER_W), jnp.int32),
        pltpu.VMEM((_G, _D), jnp.float32),
        pltpu.VMEM((_G, _D), jnp.float32),
        pltpu.VMEM((_G, _D), jnp.float32),
        pltpu.VMEM((_G, _D), jnp.float32),
        pltpu.VMEM((_G, _D), jnp.float32),
        pltpu.VMEM((_G, _D), jnp.float32),
        pltpu.VMEM((_G, _D), jnp.float32),
        pltpu.VMEM((_G, _D), jnp.float32),
        pltpu.VMEM((_G, _D), jnp.float32),
        pltpu.VMEM((_G * 64,), jnp.float32),
        pltpu.SemaphoreType.DMA,
        pltpu.SemaphoreType.DMA,
        pltpu.SemaphoreType.DMA,
    ],
)


def _out_body(a_ref, w_ref, o_ref):
    o_ref[...] = jnp.dot(a_ref[...].astype(jnp.bfloat16), w_ref[...],
                         preferred_element_type=jnp.float32)


def _outproj(attn, o_w_bf16):
    bm = 256
    return pl.pallas_call(
        _out_body,
        grid=(_P // bm,),
        in_specs=[pl.BlockSpec((bm, _D), lambda i: (i, 0)),
                  pl.BlockSpec((_D, _D), lambda i: (0, 0))],
        out_specs=pl.BlockSpec((bm, _D), lambda i: (i, 0)),
        out_shape=jax.ShapeDtypeStruct((_P, _D), jnp.float32),
    )(attn, o_w_bf16)


def kernel(hidden_states, connections, q_w, k_w_must, v_w_must, k_w_may,
           v_w_may, k_w_next, v_w_next, o_w):
    h2 = hidden_states.reshape(_P, _D)
    scale = 1.0 / (_HD ** 0.5)
    i = jnp.arange(_D)
    perm = (i % _NH) * _HD + i // _NH
    ws = [
        (q_w * scale)[:, perm].astype(jnp.bfloat16),
        k_w_must[:, perm].astype(jnp.bfloat16),
        k_w_may[:, perm].astype(jnp.bfloat16),
        k_w_next[:, perm].astype(jnp.bfloat16),
        v_w_must[:, perm].astype(jnp.bfloat16),
        v_w_may[:, perm].astype(jnp.bfloat16),
        v_w_next[:, perm].astype(jnp.bfloat16),
    ]
    halves = []
    for b in range(_B):
        q, km, ka, kn, vm, va, vn = _project_half(h2, ws, b)
        idx = connections[b].astype(jnp.int32).T
        halves.append(_sc_attn(q, idx, km, ka, kn, vm, va, vn))
    attn = jnp.concatenate(halves, axis=0)
    out = _outproj(attn, o_w[perm, :].astype(jnp.bfloat16))
    return out.reshape(_B, _S, _D)


# speedup vs baseline: 2.7351x; 1.0410x over previous
import jax
import jax.numpy as jnp
from jax import lax
from jax.experimental import pallas as pl
from jax.experimental.pallas import tpu as pltpu
from jax.experimental.pallas import tpu_sc as plsc

_B, _S, _D = 2, 4096, 1024
_NH, _HD, _NC = 16, 64, 4
_P = _B * _S
_P2 = _S
_NW = 32
_PER_W = _P2 // _NW
_G = 8
_NG = _PER_W // _G


def _proj_body(a_ref, *refs):
    w_refs, o_refs = refs[:7], refs[7:]
    a = a_ref[...].astype(jnp.bfloat16)
    for w, o in zip(w_refs, o_refs):
        o[...] = jnp.dot(a, w[...], preferred_element_type=jnp.float32)


def _project_half(h2, ws, b):
    bm = 512
    nsteps = _P2 // bm
    return pl.pallas_call(
        _proj_body,
        grid=(nsteps,),
        in_specs=[pl.BlockSpec((bm, _D), lambda i, _b=b: (i + _b * (_P2 // bm), 0))]
        + [pl.BlockSpec((_D, _D), lambda i: (0, 0))] * 7,
        out_specs=[pl.BlockSpec((bm, _D), lambda i: (i, 0))] * 7,
        out_shape=[jax.ShapeDtypeStruct((_P2, _D), jnp.float32)] * 7,
    )(h2, *ws)


def _sc_attn_body(q_hbm, idx_hbm, k0, k1, k2, v0, v1, v2, out_hbm,
                  q_v, idx_v, kb0, kb1, kb2, kb3, vb0, vb1, vb2, vb3,
                  out_v, w_scr, semq, semk, semv):
    cid = lax.axis_index("c")
    sid = lax.axis_index("s")
    wid = sid * 2 + cid
    base = wid * _PER_W

    pltpu.sync_copy(idx_hbm.at[:, pl.ds(base, _PER_W)], idx_v)

    ktabs = (k0, k1, k1, k2)
    vtabs = (v0, v1, v1, v2)
    kbufs = (kb0, kb1, kb2, kb3)
    vbufs = (vb0, vb1, vb2, vb3)

    def start_k(g):
        gb = base + g * _G
        pltpu.make_async_copy(q_hbm.at[pl.ds(gb, _G)], q_v, semq).start()
        for c in range(_NC):
            idxc = idx_v.at[c, pl.ds(g * _G, _G)]
            pltpu.make_async_copy(ktabs[c].at[idxc], kbufs[c], semk).start()

    def wait_k():
        pltpu.make_async_copy(q_hbm.at[pl.ds(0, _G)], q_v, semq).wait()
        for c in range(_NC):
            pltpu.make_async_copy(ktabs[c].at[idx_v.at[c, pl.ds(0, _G)]],
                                  kbufs[c], semk).wait()

    def start_v(g):
        for c in range(_NC):
            idxc = idx_v.at[c, pl.ds(g * _G, _G)]
            pltpu.make_async_copy(vtabs[c].at[idxc], vbufs[c], semv).start()

    def wait_v():
        for c in range(_NC):
            pltpu.make_async_copy(vtabs[c].at[idx_v.at[c, pl.ds(0, _G)]],
                                  vbufs[c], semv).wait()

    start_k(0)

    def group(g, carry):
        wait_k()
        start_v(g)

        def posk(p, pc):
            qv = q_v[p, pl.ds(0, 16)]
            acc = [qv * kbufs[c][p, pl.ds(0, 16)] for c in range(_NC)]
            for d in range(1, _HD):
                sl = pl.ds(d * 16, 16)
                qv = q_v[p, sl]
                for c in range(_NC):
                    acc[c] = acc[c] + qv * kbufs[c][p, sl]
            m = jnp.maximum(jnp.maximum(acc[0], acc[1]),
                            jnp.maximum(acc[2], acc[3]))
            es = [jnp.exp(a - m) for a in acc]
            r = 1.0 / ((es[0] + es[1]) + (es[2] + es[3]))
            for c in range(_NC):
                w_scr[pl.ds(p * 64 + c * 16, 16)] = es[c] * r
            return pc

        lax.fori_loop(0, _G, posk, 0)

        wait_v()

        @pl.when(g + 1 < _NG)
        def _():
            start_k(g + 1)

        def posv(p, pc):
            ws = [w_scr[pl.ds(p * 64 + c * 16, 16)] for c in range(_NC)]
            for d in range(_HD):
                sl = pl.ds(d * 16, 16)
                o = ws[0] * vbufs[0][p, sl]
                for c in range(1, _NC):
                    o = o + ws[c] * vbufs[c][p, sl]
                out_v[p, sl] = o
            return pc

        lax.fori_loop(0, _G, posv, 0)
        gb = base + g * _G
        pltpu.sync_copy(out_v, out_hbm.at[pl.ds(gb, _G)])
        return carry

    lax.fori_loop(0, _NG, group, 0)


_sc_attn = pl.kernel(
    _sc_attn_body,
    out_type=jax.ShapeDtypeStruct((_P2, _D), jnp.float32),
    mesh=plsc.VectorSubcoreMesh(core_axis_name="c", subcore_axis_name="s",
                                num_cores=2, num_subcores=16),
    scratch_types=[
        pltpu.VMEM((_G, _D), jnp.float32),
        pltpu.VMEM((_NC, _PER_W), jnp.int32),
        pltpu.VMEM((_G, _D), jnp.float32),
        pltpu.VMEM((_G, _D), jnp.float32),
        pltpu.VMEM((_G, _D), jnp.float32),
        pltpu.VMEM((_G, _D), jnp.float32),
        pltpu.VMEM((_G, _D), jnp.float32),
        pltpu.VMEM((_G, _D), jnp.float32),
        pltpu.VMEM((_G, _D), jnp.float32),
        pltpu.VMEM((_G, _D), jnp.float32),
        pltpu.VMEM((_G, _D), jnp.float32),
        pltpu.VMEM((_G * 64,), jnp.float32),
        pltpu.SemaphoreType.DMA,
        pltpu.SemaphoreType.DMA,
        pltpu.SemaphoreType.DMA,
    ],
)


def _out_body(a_ref, w_ref, o_ref):
    o_ref[...] = jnp.dot(a_ref[...].astype(jnp.bfloat16), w_ref[...],
                         preferred_element_type=jnp.float32)


def _outproj(attn, o_w_bf16):
    bm = 512
    return pl.pallas_call(
        _out_body,
        grid=(_P2 // bm,),
        in_specs=[pl.BlockSpec((bm, _D), lambda i: (i, 0)),
                  pl.BlockSpec((_D, _D), lambda i: (0, 0))],
        out_specs=pl.BlockSpec((bm, _D), lambda i: (i, 0)),
        out_shape=jax.ShapeDtypeStruct((_P2, _D), jnp.float32),
    )(attn, o_w_bf16)


def kernel(hidden_states, connections, q_w, k_w_must, v_w_must, k_w_may,
           v_w_may, k_w_next, v_w_next, o_w):
    h2 = hidden_states.reshape(_P, _D)
    scale = 1.0 / (_HD ** 0.5)
    i = jnp.arange(_D)
    perm = (i % _NH) * _HD + i // _NH
    ws = [
        (q_w * scale)[:, perm].astype(jnp.bfloat16),
        k_w_must[:, perm].astype(jnp.bfloat16),
        k_w_may[:, perm].astype(jnp.bfloat16),
        k_w_next[:, perm].astype(jnp.bfloat16),
        v_w_must[:, perm].astype(jnp.bfloat16),
        v_w_may[:, perm].astype(jnp.bfloat16),
        v_w_next[:, perm].astype(jnp.bfloat16),
    ]
    o_w_p = o_w[perm, :].astype(jnp.bfloat16)
    outs = []
    for b in range(_B):
        q, km, ka, kn, vm, va, vn = _project_half(h2, ws, b)
        idx = connections[b].astype(jnp.int32).T
        attn = _sc_attn(q, idx, km, ka, kn, vm, va, vn)
        outs.append(_outproj(attn, o_w_p))
    return jnp.stack(outs, axis=0)
